# baseline (speedup 1.0000x reference)
import functools

import jax
import jax.numpy as jnp
from jax import lax
from jax.experimental import pallas as pl
from jax.experimental.pallas import tpu as pltpu
from jax.experimental.pallas import tpu_sc as plsc

NN = 10000
NP = 10240
NE = 160000
EP = 163840
IN_CH = 256
H2 = 512
HID = 256
CW = 128
NCORES = 2
NTILES = 16
SLAB = NP // NTILES
WIN = 128
AWIN = 80
NBUF = 4
NIF = NBUF - 1
STG = 16

def _mesh():
    return plsc.VectorSubcoreMesh(core_axis_name="c", subcore_axis_name="s")


def _deg_kernel(dst2d, ones):
    nwin_total = EP // WIN
    nwin_core = nwin_total // NCORES
    nwin_tile = nwin_core // NTILES

    @functools.partial(
        pl.kernel,
        out_type=jax.ShapeDtypeStruct((NCORES, NP), jnp.float32),
        mesh=_mesh(),
        scratch_types=[
            pltpu.VMEM_SHARED((NP,), jnp.float32),
            pltpu.VMEM((nwin_tile, WIN), jnp.int32),
            pltpu.VMEM((WIN,), jnp.float32),
            pltpu.VMEM((SLAB,), jnp.float32),
            pltpu.SemaphoreType.DMA,
        ],
    )
    def k(dst_hbm, ones_hbm, deg_hbm, deg_sp, idx_v, ones_v, zslab_v, sem):
        core = lax.axis_index("c")
        sub = lax.axis_index("s")
        row0 = (core * NTILES + sub) * nwin_tile
        pltpu.sync_copy(dst_hbm.at[pl.ds(row0, nwin_tile)], idx_v)
        pltpu.sync_copy(ones_hbm, ones_v)
        zval = jnp.zeros((16,), jnp.float32)

        @pl.loop(0, SLAB // 16)
        def _(i):
            zslab_v[pl.ds(i * 16, 16)] = zval

        pltpu.sync_copy(zslab_v, deg_sp.at[pl.ds(sub * SLAB, SLAB)])
        plsc.subcore_barrier()

        @pl.loop(0, nwin_tile)
        def _(w):
            pltpu.sync_copy(ones_v, deg_sp.at[idx_v.at[w]], add=True)

        plsc.subcore_barrier()
        pltpu.sync_copy(deg_sp.at[pl.ds(sub * SLAB, SLAB)],
                        deg_hbm.at[core].at[pl.ds(sub * SLAB, SLAB)])

    return k(dst2d, ones)


def _agg_kernel(p, src2d, dst2d, nc):
    cpc = nc // NCORES
    nwin_tile = (EP // AWIN) // NTILES
    nstg = nwin_tile // STG

    @functools.partial(
        pl.kernel,
        out_type=jax.ShapeDtypeStruct((nc, NP, CW), jnp.float32),
        mesh=_mesh(),
        scratch_types=[
            pltpu.VMEM_SHARED((NP, CW), jnp.float32),
            pltpu.VMEM((STG, AWIN), jnp.int32),
            pltpu.VMEM((STG, AWIN), jnp.int32),
        ]
        + [pltpu.VMEM((AWIN, CW), jnp.float32) for _ in range(NBUF)]
        + [pltpu.SemaphoreType.DMA for _ in range(2 * NBUF)],
    )
    def k(p_hbm, src_hbm, dst_hbm, out_hbm, agg_sp, src_v, dst_v,
          *bufs_and_sems):
        rows = bufs_and_sems[:NBUF]
        gsems = bufs_and_sems[NBUF:2 * NBUF]
        ssems = bufs_and_sems[2 * NBUF:]
        core = lax.axis_index("c")
        sub = lax.axis_index("s")
        row0 = sub * nwin_tile

        def issue_gather(p_c, w, b):
            pltpu.async_copy(p_c.at[src_v.at[w]], rows[b], gsems[b])

        def wait_gather(p_c, w, b):
            pltpu.make_async_copy(p_c.at[src_v.at[w]], rows[b],
                                  gsems[b]).wait()

        def issue_scatter(w, b):
            pltpu.async_copy(rows[b], agg_sp.at[dst_v.at[w]],
                             ssems[b], add=True)

        def wait_scatter(w, b):
            pltpu.make_async_copy(rows[b], agg_sp.at[dst_v.at[w]],
                                  ssems[b]).wait()

        def step(p_c, w, b, prev_wait=True, prefetch=True):
            wait_gather(p_c, w, b)
            issue_scatter(w, b)
            if prev_wait:
                wait_scatter(w - 1, (b - 1) % NBUF)
            if prefetch:
                issue_gather(p_c, w + NIF, (b + NIF) % NBUF)

        for j in range(cpc):
            chunk = core * cpc + j
            p_c = p_hbm.at[chunk]
            pltpu.sync_copy(p_c.at[pl.ds(sub * SLAB, SLAB)],
                            agg_sp.at[pl.ds(sub * SLAB, SLAB)])
            plsc.subcore_barrier()

            for h in range(nstg):
                pltpu.sync_copy(src_hbm.at[pl.ds(row0 + h * STG, STG)],
                                src_v)
                pltpu.sync_copy(dst_hbm.at[pl.ds(row0 + h * STG, STG)],
                                dst_v)

                for i in range(NIF):
                    issue_gather(p_c, i, i)
                for w in range(NBUF):
                    step(p_c, w, w, prev_wait=(w > 0))

                @pl.loop(1, STG // NBUF - 1)
                def _(t):
                    for b in range(NBUF):
                        step(p_c, t * NBUF + b, b)

                for b in range(NBUF):
                    w = STG - NBUF + b
                    step(p_c, w, b, prefetch=(w + NIF < STG))
                wait_scatter(STG - 1, (STG - 1) % NBUF)

            plsc.subcore_barrier()
            pltpu.sync_copy(agg_sp.at[pl.ds(sub * SLAB, SLAB)],
                            out_hbm.at[chunk].at[pl.ds(sub * SLAB, SLAB)])
            if j + 1 < cpc:
                plsc.subcore_barrier()

    return k(p, src2d, dst2d)


RB = 1000


def _t1_body(d0_ref, d1_ref, x_ref, out_ref):
    dinv = lax.rsqrt(d0_ref[...] + d1_ref[...] + 1.0)
    xs = x_ref[...] * dinv
    for c in range(IN_CH // CW):
        out_ref[c, :, :] = xs[:, c * CW:(c + 1) * CW]


def _t2_body(d0_ref, d1_ref, agg_ref, w1_ref, b1_ref, w2_ref, out_ref):
    dinv = lax.rsqrt(d0_ref[...] + d1_ref[...] + 1.0)
    ax = jnp.concatenate([agg_ref[c, :, :] for c in range(IN_CH // CW)],
                         axis=1)
    g1 = lax.dot_general((dinv * ax).astype(jnp.bfloat16),
                         w1_ref[...].astype(jnp.bfloat16),
                         (((1,), (0,)), ((), ())),
                         preferred_element_type=jnp.float32)
    h = jax.nn.relu(g1 + b1_ref[...])
    p2 = lax.dot_general((dinv * h).astype(jnp.bfloat16),
                         w2_ref[...].astype(jnp.bfloat16),
                         (((1,), (0,)), ((), ())),
                         preferred_element_type=jnp.float32)
    for q in range(HID // CW):
        out_ref[q, :, :] = p2[:, q * CW:(q + 1) * CW]


def _t3_body(d0_ref, d1_ref, agg_ref, b2_ref, out_ref):
    dinv = lax.rsqrt(d0_ref[...] + d1_ref[...] + 1.0)
    z = jnp.concatenate([agg_ref[q, :, :] for q in range(HID // CW)], axis=1)
    out_ref[...] = z * dinv + b2_ref[...]


def _dvec_spec():
    return pl.BlockSpec((RB, 1), lambda i: (i, 0))


def kernel(x, edge_index, W1, b1, W2, b2):
    src = edge_index[0]
    dst = edge_index[1]
    pad = (jnp.arange(EP - NE, dtype=jnp.int32) % (NP - NN)) + NN
    srcp = jnp.concatenate([src, pad])
    dstp = jnp.concatenate([dst, pad])
    src2d = srcp.reshape(EP // AWIN, AWIN)
    dst2d = dstp.reshape(EP // AWIN, AWIN)
    dst2d_deg = dstp.reshape(EP // WIN, WIN)
    ones = jnp.ones((WIN,), jnp.float32)

    deg_parts = _deg_kernel(dst2d_deg, ones)
    d0 = deg_parts[0].reshape(NP, 1)
    d1 = deg_parts[1].reshape(NP, 1)

    grid = (NN // RB,)
    xs = pl.pallas_call(
        _t1_body,
        grid=grid,
        in_specs=[
            _dvec_spec(), _dvec_spec(),
            pl.BlockSpec((RB, IN_CH), lambda i: (i, 0)),
        ],
        out_specs=pl.BlockSpec((IN_CH // CW, RB, CW), lambda i: (0, i, 0)),
        out_shape=jax.ShapeDtypeStruct((IN_CH // CW, NP, CW), jnp.float32),
    )(d0, d1, x)

    agg1 = _agg_kernel(xs, src2d, dst2d, IN_CH // CW)

    p2 = pl.pallas_call(
        _t2_body,
        grid=grid,
        in_specs=[
            _dvec_spec(), _dvec_spec(),
            pl.BlockSpec((IN_CH // CW, RB, CW), lambda i: (0, i, 0)),
            pl.BlockSpec((IN_CH, H2), lambda i: (0, 0)),
            pl.BlockSpec((1, H2), lambda i: (0, 0)),
            pl.BlockSpec((H2, HID), lambda i: (0, 0)),
        ],
        out_specs=pl.BlockSpec((HID // CW, RB, CW), lambda i: (0, i, 0)),
        out_shape=jax.ShapeDtypeStruct((HID // CW, NP, CW), jnp.float32),
    )(d0, d1, agg1, W1, b1.reshape(1, H2), W2)

    agg2 = _agg_kernel(p2, src2d, dst2d, HID // CW)

    z = pl.pallas_call(
        _t3_body,
        grid=grid,
        in_specs=[
            _dvec_spec(), _dvec_spec(),
            pl.BlockSpec((HID // CW, RB, CW), lambda i: (0, i, 0)),
            pl.BlockSpec((1, HID), lambda i: (0, 0)),
        ],
        out_specs=pl.BlockSpec((RB, HID), lambda i: (i, 0)),
        out_shape=jax.ShapeDtypeStruct((NN, HID), jnp.float32),
    )(d0, d1, agg2, b2.reshape(1, HID))

    return z


# speedup vs baseline: 1.0572x; 1.0572x over previous
import functools

import jax
import jax.numpy as jnp
from jax import lax
from jax.experimental import pallas as pl
from jax.experimental.pallas import tpu as pltpu
from jax.experimental.pallas import tpu_sc as plsc

NN = 10000
NP = 10240
NE = 160000
EP = 163840
IN_CH = 256
H2 = 512
HID = 256
CW = 128
NCORES = 2
NTILES = 16
SLAB = NP // NTILES
WIN = 128
AWIN = 64
NBUF = 4
NIF = NBUF - 1
STG = 40

def _mesh():
    return plsc.VectorSubcoreMesh(core_axis_name="c", subcore_axis_name="s")


def _deg_kernel(dst2d, ones):
    nwin_total = EP // WIN
    nwin_core = nwin_total // NCORES
    nwin_tile = nwin_core // NTILES

    @functools.partial(
        pl.kernel,
        out_type=jax.ShapeDtypeStruct((NCORES, NP), jnp.float32),
        mesh=_mesh(),
        scratch_types=[
            pltpu.VMEM_SHARED((NP,), jnp.float32),
            pltpu.VMEM((nwin_tile, WIN), jnp.int32),
            pltpu.VMEM((WIN,), jnp.float32),
            pltpu.VMEM((SLAB,), jnp.float32),
            pltpu.SemaphoreType.DMA,
        ],
    )
    def k(dst_hbm, ones_hbm, deg_hbm, deg_sp, idx_v, ones_v, zslab_v, sem):
        core = lax.axis_index("c")
        sub = lax.axis_index("s")
        row0 = (core * NTILES + sub) * nwin_tile
        pltpu.sync_copy(dst_hbm.at[pl.ds(row0, nwin_tile)], idx_v)
        pltpu.sync_copy(ones_hbm, ones_v)
        zval = jnp.zeros((16,), jnp.float32)

        @pl.loop(0, SLAB // 16)
        def _(i):
            zslab_v[pl.ds(i * 16, 16)] = zval

        pltpu.sync_copy(zslab_v, deg_sp.at[pl.ds(sub * SLAB, SLAB)])
        plsc.subcore_barrier()

        @pl.loop(0, nwin_tile)
        def _(w):
            pltpu.sync_copy(ones_v, deg_sp.at[idx_v.at[w]], add=True)

        plsc.subcore_barrier()
        pltpu.sync_copy(deg_sp.at[pl.ds(sub * SLAB, SLAB)],
                        deg_hbm.at[core].at[pl.ds(sub * SLAB, SLAB)])

    return k(dst2d, ones)


def _agg_kernel(p, src2d, dst2d, nc):
    cpc = nc // NCORES
    nwin_tile = (EP // AWIN) // NTILES
    nstg = nwin_tile // STG

    @functools.partial(
        pl.kernel,
        out_type=jax.ShapeDtypeStruct((nc, NP, CW), jnp.float32),
        mesh=_mesh(),
        scratch_types=[
            pltpu.VMEM_SHARED((NP, CW), jnp.float32),
            pltpu.VMEM((STG, AWIN), jnp.int32),
            pltpu.VMEM((STG, AWIN), jnp.int32),
        ]
        + [pltpu.VMEM((AWIN, CW), jnp.float32) for _ in range(NBUF)]
        + [pltpu.SemaphoreType.DMA for _ in range(2 * NBUF)],
    )
    def k(p_hbm, src_hbm, dst_hbm, out_hbm, agg_sp, src_v, dst_v,
          *bufs_and_sems):
        rows = bufs_and_sems[:NBUF]
        gsems = bufs_and_sems[NBUF:2 * NBUF]
        ssems = bufs_and_sems[2 * NBUF:]
        core = lax.axis_index("c")
        sub = lax.axis_index("s")
        row0 = sub * nwin_tile

        def issue_gather(p_c, w, b):
            pltpu.async_copy(p_c.at[src_v.at[w]], rows[b], gsems[b])

        def wait_gather(p_c, w, b):
            pltpu.make_async_copy(p_c.at[src_v.at[w]], rows[b],
                                  gsems[b]).wait()

        def issue_scatter(w, b):
            pltpu.async_copy(rows[b], agg_sp.at[dst_v.at[w]],
                             ssems[b], add=True)

        def wait_scatter(w, b):
            pltpu.make_async_copy(rows[b], agg_sp.at[dst_v.at[w]],
                                  ssems[b]).wait()

        def step(p_c, w, b, prev_wait=True, prefetch=True):
            wait_gather(p_c, w, b)
            issue_scatter(w, b)
            if prev_wait:
                wait_scatter(w - 1, (b - 1) % NBUF)
            if prefetch:
                issue_gather(p_c, w + NIF, (b + NIF) % NBUF)

        for j in range(cpc):
            chunk = core * cpc + j
            p_c = p_hbm.at[chunk]
            pltpu.sync_copy(p_c.at[pl.ds(sub * SLAB, SLAB)],
                            agg_sp.at[pl.ds(sub * SLAB, SLAB)])
            plsc.subcore_barrier()

            for h in range(nstg):
                pltpu.sync_copy(src_hbm.at[pl.ds(row0 + h * STG, STG)],
                                src_v)
                pltpu.sync_copy(dst_hbm.at[pl.ds(row0 + h * STG, STG)],
                                dst_v)

                for i in range(NIF):
                    issue_gather(p_c, i, i)
                for w in range(NBUF):
                    step(p_c, w, w, prev_wait=(w > 0))

                @pl.loop(1, STG // NBUF - 1)
                def _(t):
                    for b in range(NBUF):
                        step(p_c, t * NBUF + b, b)

                for b in range(NBUF):
                    w = STG - NBUF + b
                    step(p_c, w, b, prefetch=(w + NIF < STG))
                wait_scatter(STG - 1, (STG - 1) % NBUF)

            plsc.subcore_barrier()
            pltpu.sync_copy(agg_sp.at[pl.ds(sub * SLAB, SLAB)],
                            out_hbm.at[chunk].at[pl.ds(sub * SLAB, SLAB)])
            if j + 1 < cpc:
                plsc.subcore_barrier()

    return k(p, src2d, dst2d)


RB = 1000


def _t1_body(d0_ref, d1_ref, x_ref, out_ref):
    dinv = lax.rsqrt(d0_ref[...] + d1_ref[...] + 1.0)
    xs = x_ref[...] * dinv
    for c in range(IN_CH // CW):
        out_ref[c, :, :] = xs[:, c * CW:(c + 1) * CW]


def _t2_body(d0_ref, d1_ref, agg_ref, w1_ref, b1_ref, w2_ref, out_ref):
    dinv = lax.rsqrt(d0_ref[...] + d1_ref[...] + 1.0)
    ax = jnp.concatenate([agg_ref[c, :, :] for c in range(IN_CH // CW)],
                         axis=1)
    g1 = lax.dot_general((dinv * ax).astype(jnp.bfloat16),
                         w1_ref[...].astype(jnp.bfloat16),
                         (((1,), (0,)), ((), ())),
                         preferred_element_type=jnp.float32)
    h = jax.nn.relu(g1 + b1_ref[...])
    p2 = lax.dot_general((dinv * h).astype(jnp.bfloat16),
                         w2_ref[...].astype(jnp.bfloat16),
                         (((1,), (0,)), ((), ())),
                         preferred_element_type=jnp.float32)
    for q in range(HID // CW):
        out_ref[q, :, :] = p2[:, q * CW:(q + 1) * CW]


def _t3_body(d0_ref, d1_ref, agg_ref, b2_ref, out_ref):
    dinv = lax.rsqrt(d0_ref[...] + d1_ref[...] + 1.0)
    z = jnp.concatenate([agg_ref[q, :, :] for q in range(HID // CW)], axis=1)
    out_ref[...] = z * dinv + b2_ref[...]


def _dvec_spec():
    return pl.BlockSpec((RB, 1), lambda i: (i, 0))


def kernel(x, edge_index, W1, b1, W2, b2):
    src = edge_index[0]
    dst = edge_index[1]
    pad = (jnp.arange(EP - NE, dtype=jnp.int32) % (NP - NN)) + NN
    srcp = jnp.concatenate([src, pad])
    dstp = jnp.concatenate([dst, pad])
    src2d = srcp.reshape(EP // AWIN, AWIN)
    dst2d = dstp.reshape(EP // AWIN, AWIN)
    dst2d_deg = dstp.reshape(EP // WIN, WIN)
    ones = jnp.ones((WIN,), jnp.float32)

    deg_parts = _deg_kernel(dst2d_deg, ones)
    d0 = deg_parts[0].reshape(NP, 1)
    d1 = deg_parts[1].reshape(NP, 1)

    grid = (NN // RB,)
    xs = pl.pallas_call(
        _t1_body,
        grid=grid,
        in_specs=[
            _dvec_spec(), _dvec_spec(),
            pl.BlockSpec((RB, IN_CH), lambda i: (i, 0)),
        ],
        out_specs=pl.BlockSpec((IN_CH // CW, RB, CW), lambda i: (0, i, 0)),
        out_shape=jax.ShapeDtypeStruct((IN_CH // CW, NP, CW), jnp.float32),
    )(d0, d1, x)

    agg1 = _agg_kernel(xs, src2d, dst2d, IN_CH // CW)

    p2 = pl.pallas_call(
        _t2_body,
        grid=grid,
        in_specs=[
            _dvec_spec(), _dvec_spec(),
            pl.BlockSpec((IN_CH // CW, RB, CW), lambda i: (0, i, 0)),
            pl.BlockSpec((IN_CH, H2), lambda i: (0, 0)),
            pl.BlockSpec((1, H2), lambda i: (0, 0)),
            pl.BlockSpec((H2, HID), lambda i: (0, 0)),
        ],
        out_specs=pl.BlockSpec((HID // CW, RB, CW), lambda i: (0, i, 0)),
        out_shape=jax.ShapeDtypeStruct((HID // CW, NP, CW), jnp.float32),
    )(d0, d1, agg1, W1, b1.reshape(1, H2), W2)

    agg2 = _agg_kernel(p2, src2d, dst2d, HID // CW)

    z = pl.pallas_call(
        _t3_body,
        grid=grid,
        in_specs=[
            _dvec_spec(), _dvec_spec(),
            pl.BlockSpec((HID // CW, RB, CW), lambda i: (0, i, 0)),
            pl.BlockSpec((1, HID), lambda i: (0, 0)),
        ],
        out_specs=pl.BlockSpec((RB, HID), lambda i: (i, 0)),
        out_shape=jax.ShapeDtypeStruct((NN, HID), jnp.float32),
    )(d0, d1, agg2, b2.reshape(1, HID))

    return z
